# baseline (speedup 1.0000x reference)
import functools
import math

import jax
import jax.numpy as jnp
from jax import lax
from jax.experimental import pallas as pl
from jax.experimental.pallas import tpu as pltpu
from jax.experimental.pallas import tpu_sc as plsc

N = 10000
F = 128
E = 320000
_NPAD = 10240
ALPHA = 0.1
THETA = 0.5
EPS = 1e-5

_ROW_BLOCK = 1000
_GRID = N // _ROW_BLOCK


def _x0_body(x_ref, w_ref, b_ref, o_ref):
  acc = lax.dot_general(x_ref[...], w_ref[...], (((1,), (1,)), ((), ())),
                        preferred_element_type=jnp.float32)
  o_ref[...] = jnp.maximum(acc + b_ref[...], 0.0)


def _compute_x0(x, lin_w, lin_b2):
  return pl.pallas_call(
      _x0_body,
      grid=(_GRID,),
      in_specs=[
          pl.BlockSpec((_ROW_BLOCK, F), lambda i: (i, 0)),
          pl.BlockSpec((F, F), lambda i: (0, 0)),
          pl.BlockSpec((1, F), lambda i: (0, 0)),
      ],
      out_specs=pl.BlockSpec((_ROW_BLOCK, F), lambda i: (i, 0)),
      out_shape=jax.ShapeDtypeStruct((N, F), jnp.float32),
  )(x, lin_w, lin_b2)


def _seg_dense_body(m_ref, x0_ref, w_ref, h_ref, s_ref, *, beta):
  m = m_ref[0] + m_ref[1]
  t = (1.0 - ALPHA) * m + ALPHA * x0_ref[...]
  h = (1.0 - beta) * t + beta * jnp.dot(
      t, w_ref[...], preferred_element_type=jnp.float32)
  h_ref[...] = h

  @pl.when(pl.program_id(0) == 0)
  def _():
    s_ref[...] = jnp.zeros_like(s_ref)

  s_ref[0:1, :] = s_ref[0:1, :] + jnp.sum(h, axis=0, keepdims=True)
  s_ref[1:2, :] = s_ref[1:2, :] + jnp.sum(h * h, axis=0, keepdims=True)


def _bn_body(h_ref, s_ref, g_ref, b_ref, o_ref):
  h = h_ref[...]
  mean = s_ref[0:1, :] * (1.0 / N)
  var = s_ref[1:2, :] * (1.0 / N) - mean * mean
  inv = lax.rsqrt(var + EPS)
  o_ref[...] = jnp.maximum((h - mean) * inv * g_ref[...] + b_ref[...], 0.0)


def _dense_layer(mpart, x0, w, g2, b2, beta):
  h, stats = pl.pallas_call(
      functools.partial(_seg_dense_body, beta=beta),
      grid=(_GRID,),
      in_specs=[
          pl.BlockSpec((2, _ROW_BLOCK, F), lambda i: (0, i, 0)),
          pl.BlockSpec((_ROW_BLOCK, F), lambda i: (i, 0)),
          pl.BlockSpec((F, F), lambda i: (0, 0)),
      ],
      out_specs=[
          pl.BlockSpec((_ROW_BLOCK, F), lambda i: (i, 0)),
          pl.BlockSpec((8, F), lambda i: (0, 0)),
      ],
      out_shape=[
          jax.ShapeDtypeStruct((N, F), jnp.float32),
          jax.ShapeDtypeStruct((8, F), jnp.float32),
      ],
  )(mpart, x0, w)
  return pl.pallas_call(
      _bn_body,
      grid=(_GRID,),
      in_specs=[
          pl.BlockSpec((_ROW_BLOCK, F), lambda i: (i, 0)),
          pl.BlockSpec((8, F), lambda i: (0, 0)),
          pl.BlockSpec((1, F), lambda i: (0, 0)),
          pl.BlockSpec((1, F), lambda i: (0, 0)),
      ],
      out_specs=pl.BlockSpec((_ROW_BLOCK, F), lambda i: (i, 0)),
      out_shape=jax.ShapeDtypeStruct((N, F), jnp.float32),
  )(h, stats, g2, b2)


_CHUNK = 128
_NITER = 80
_EPAD = 32 * _NITER * _CHUNK


def _make_segsum():
  info = plsc.get_sparse_core_info()
  nc, ns = info.num_cores, info.num_subcores
  nw = nc * ns
  chunk = _CHUNK
  niter = _NITER
  npad = _NPAD
  rpt = npad // ns

  mesh = plsc.VectorSubcoreMesh(core_axis_name="c", subcore_axis_name="s")

  @functools.partial(
      pl.kernel,
      out_type=jax.ShapeDtypeStruct((nc, npad, F), jnp.float32),
      mesh=mesh,
      scratch_types=[
          pltpu.VMEM((2, chunk), jnp.int32),
          pltpu.VMEM((2, chunk), jnp.int32),
          pltpu.VMEM((chunk, F), jnp.float32),
          pltpu.VMEM((chunk, F), jnp.float32),
          pltpu.VMEM_SHARED((npad, F), jnp.float32),
          pltpu.SemaphoreType.DMA,
          pltpu.SemaphoreType.DMA,
          pltpu.SemaphoreType.DMA,
          pltpu.SemaphoreType.DMA,
      ],
  )
  def segsum(h_hbm, eidx_hbm, out_hbm, eidx0, eidx1, rows0, rows1, acc,
             sema, semb, semi0, semi1):
    cid = lax.axis_index("c")
    sid = lax.axis_index("s")
    wid = sid * nc + (1 - cid)

    def zero_body(i, carry):
      rows0[i // 8, pl.ds((i % 8) * 16, 16)] = jnp.zeros((16,), jnp.float32)
      return carry

    lax.fori_loop(0, chunk * 8, zero_body, 0)
    row0 = sid * rpt
    for j in range(rpt // chunk):
      pltpu.sync_copy(rows0, acc.at[pl.ds(row0 + j * chunk, chunk)])
    rem = rpt % chunk
    if rem:
      pltpu.sync_copy(rows0.at[pl.ds(0, rem)],
                      acc.at[pl.ds(row0 + rpt - rem, rem)])
    plsc.subcore_barrier()

    pltpu.sync_copy(eidx_hbm.at[wid, 0], eidx0)
    pltpu.async_copy(h_hbm.at[eidx0.at[0]], rows0, sema)
    pltpu.sync_copy(eidx_hbm.at[wid, 1], eidx1)
    pltpu.async_copy(h_hbm.at[eidx1.at[0]], rows1, semb)

    def pair_body(p, carry):
      i = p * 2
      pltpu.make_async_copy(h_hbm.at[eidx0.at[0]], rows0, sema).wait()
      pltpu.sync_copy(rows0, acc.at[eidx0.at[1]], add=True)
      pltpu.async_copy(eidx_hbm.at[wid, i + 2], eidx0, semi0)
      pltpu.make_async_copy(h_hbm.at[eidx1.at[0]], rows1, semb).wait()
      pltpu.sync_copy(rows1, acc.at[eidx1.at[1]], add=True)
      pltpu.async_copy(eidx_hbm.at[wid, i + 3], eidx1, semi1)
      pltpu.make_async_copy(eidx_hbm.at[wid, i + 2], eidx0, semi0).wait()
      pltpu.async_copy(h_hbm.at[eidx0.at[0]], rows0, sema)
      pltpu.make_async_copy(eidx_hbm.at[wid, i + 3], eidx1, semi1).wait()
      pltpu.async_copy(h_hbm.at[eidx1.at[0]], rows1, semb)
      return carry

    lax.fori_loop(0, niter // 2 - 1, pair_body, 0)

    pltpu.make_async_copy(h_hbm.at[eidx0.at[0]], rows0, sema).wait()
    pltpu.sync_copy(rows0, acc.at[eidx0.at[1]], add=True)
    pltpu.make_async_copy(h_hbm.at[eidx1.at[0]], rows1, semb).wait()
    pltpu.sync_copy(rows1, acc.at[eidx1.at[1]], add=True)

    plsc.subcore_barrier()
    pltpu.sync_copy(acc.at[pl.ds(row0, rpt)],
                    out_hbm.at[cid, pl.ds(row0, rpt)])

  return segsum


@functools.cache
def _segsum_fn():
  return _make_segsum()


def _segsum(h, eidx):
  return _segsum_fn()(h, eidx)


def kernel(x, edge_index, lin_w, lin_b, conv_w1, conv_w2, bn_gamma, bn_beta):
  pad = _EPAD - E
  src = jnp.concatenate(
      [edge_index[0], jnp.zeros((pad,), jnp.int32)]).reshape(32, _NITER, _CHUNK)
  dst = jnp.concatenate(
      [edge_index[1], jnp.full((pad,), N + 16, jnp.int32)]).reshape(
          32, _NITER, _CHUNK)
  eidx = jnp.stack([src, dst], axis=2)
  lin_b2 = lin_b.reshape(1, F)
  g2 = bn_gamma.reshape(1, F)
  b2 = bn_beta.reshape(1, F)

  x0 = _compute_x0(x, lin_w, lin_b2)
  h = x0
  for layer, w in enumerate((conv_w1, conv_w2), start=1):
    beta = float(math.log(THETA / layer + 1.0))
    mpart = _segsum(h, eidx)
    h = _dense_layer(mpart, x0, w, g2, b2, beta)
  return h


# speedup vs baseline: 1.0102x; 1.0102x over previous
import functools
import math

import jax
import jax.numpy as jnp
from jax import lax
from jax.experimental import pallas as pl
from jax.experimental.pallas import tpu as pltpu
from jax.experimental.pallas import tpu_sc as plsc

N = 10000
F = 128
E = 320000
_NPAD = 10240
ALPHA = 0.1
THETA = 0.5
EPS = 1e-5

_ROW_BLOCK = 1000
_GRID = N // _ROW_BLOCK


def _x0_body(x_ref, w_ref, b_ref, o_ref):
  acc = lax.dot_general(x_ref[...], w_ref[...], (((1,), (1,)), ((), ())),
                        preferred_element_type=jnp.float32)
  o_ref[...] = jnp.maximum(acc + b_ref[...], 0.0)


def _compute_x0(x, lin_w, lin_b2):
  return pl.pallas_call(
      _x0_body,
      grid=(_GRID,),
      in_specs=[
          pl.BlockSpec((_ROW_BLOCK, F), lambda i: (i, 0)),
          pl.BlockSpec((F, F), lambda i: (0, 0)),
          pl.BlockSpec((1, F), lambda i: (0, 0)),
      ],
      out_specs=pl.BlockSpec((_ROW_BLOCK, F), lambda i: (i, 0)),
      out_shape=jax.ShapeDtypeStruct((N, F), jnp.float32),
  )(x, lin_w, lin_b2)


def _seg_dense_body(m_ref, x0_ref, w_ref, h_ref, s_ref, *, beta):
  m = m_ref[0] + m_ref[1]
  t = (1.0 - ALPHA) * m + ALPHA * x0_ref[...]
  h = (1.0 - beta) * t + beta * jnp.dot(
      t, w_ref[...], preferred_element_type=jnp.float32)
  h_ref[...] = h

  @pl.when(pl.program_id(0) == 0)
  def _():
    s_ref[...] = jnp.zeros_like(s_ref)

  s_ref[0:1, :] = s_ref[0:1, :] + jnp.sum(h, axis=0, keepdims=True)
  s_ref[1:2, :] = s_ref[1:2, :] + jnp.sum(h * h, axis=0, keepdims=True)


def _bn_body(h_ref, s_ref, g_ref, b_ref, o_ref):
  h = h_ref[...]
  mean = s_ref[0:1, :] * (1.0 / N)
  var = s_ref[1:2, :] * (1.0 / N) - mean * mean
  inv = lax.rsqrt(var + EPS)
  o_ref[...] = jnp.maximum((h - mean) * inv * g_ref[...] + b_ref[...], 0.0)


def _dense_layer(mpart, x0, w, g2, b2, beta):
  h, stats = pl.pallas_call(
      functools.partial(_seg_dense_body, beta=beta),
      grid=(_GRID,),
      in_specs=[
          pl.BlockSpec((2, _ROW_BLOCK, F), lambda i: (0, i, 0)),
          pl.BlockSpec((_ROW_BLOCK, F), lambda i: (i, 0)),
          pl.BlockSpec((F, F), lambda i: (0, 0)),
      ],
      out_specs=[
          pl.BlockSpec((_ROW_BLOCK, F), lambda i: (i, 0)),
          pl.BlockSpec((8, F), lambda i: (0, 0)),
      ],
      out_shape=[
          jax.ShapeDtypeStruct((N, F), jnp.float32),
          jax.ShapeDtypeStruct((8, F), jnp.float32),
      ],
  )(mpart, x0, w)
  return pl.pallas_call(
      _bn_body,
      grid=(_GRID,),
      in_specs=[
          pl.BlockSpec((_ROW_BLOCK, F), lambda i: (i, 0)),
          pl.BlockSpec((8, F), lambda i: (0, 0)),
          pl.BlockSpec((1, F), lambda i: (0, 0)),
          pl.BlockSpec((1, F), lambda i: (0, 0)),
      ],
      out_specs=pl.BlockSpec((_ROW_BLOCK, F), lambda i: (i, 0)),
      out_shape=jax.ShapeDtypeStruct((N, F), jnp.float32),
  )(h, stats, g2, b2)


_CHUNK = 128
_NITER = 80
_EPAD = 32 * _NITER * _CHUNK


def _make_segsum():
  info = plsc.get_sparse_core_info()
  nc, ns = info.num_cores, info.num_subcores
  nw = nc * ns
  chunk = _CHUNK
  niter = _NITER
  npad = _NPAD
  rpt = npad // ns

  mesh = plsc.VectorSubcoreMesh(core_axis_name="c", subcore_axis_name="s")

  @functools.partial(
      pl.kernel,
      out_type=jax.ShapeDtypeStruct((nc, npad, F), jnp.float32),
      mesh=mesh,
      scratch_types=[
          pltpu.VMEM((2, chunk), jnp.int32),
          pltpu.VMEM((2, chunk), jnp.int32),
          pltpu.VMEM((chunk, F), jnp.float32),
          pltpu.VMEM((chunk, F), jnp.float32),
          pltpu.VMEM_SHARED((npad, F), jnp.float32),
          pltpu.SemaphoreType.DMA,
          pltpu.SemaphoreType.DMA,
          pltpu.SemaphoreType.DMA,
          pltpu.SemaphoreType.DMA,
      ],
  )
  def segsum(h_hbm, eidx_hbm, out_hbm, eidx0, eidx1, rows0, rows1, acc,
             sema, semb, semi0, semi1):
    cid = lax.axis_index("c")
    sid = lax.axis_index("s")
    wid = sid * nc + cid

    def zero_body(i, carry):
      rows0[i // 8, pl.ds((i % 8) * 16, 16)] = jnp.zeros((16,), jnp.float32)
      return carry

    lax.fori_loop(0, chunk * 8, zero_body, 0)
    row0 = sid * rpt
    for j in range(rpt // chunk):
      pltpu.sync_copy(rows0, acc.at[pl.ds(row0 + j * chunk, chunk)])
    rem = rpt % chunk
    if rem:
      pltpu.sync_copy(rows0.at[pl.ds(0, rem)],
                      acc.at[pl.ds(row0 + rpt - rem, rem)])
    plsc.subcore_barrier()

    pltpu.sync_copy(eidx_hbm.at[wid, 0], eidx0)
    pltpu.async_copy(h_hbm.at[eidx0.at[0]], rows0, sema)
    pltpu.sync_copy(eidx_hbm.at[wid, 1], eidx1)
    pltpu.async_copy(h_hbm.at[eidx1.at[0]], rows1, semb)

    def pair_body(p, carry):
      i = p * 2
      pltpu.make_async_copy(h_hbm.at[eidx0.at[0]], rows0, sema).wait()
      pltpu.sync_copy(rows0, acc.at[eidx0.at[1]], add=True)
      pltpu.async_copy(eidx_hbm.at[wid, i + 2], eidx0, semi0)
      pltpu.make_async_copy(h_hbm.at[eidx1.at[0]], rows1, semb).wait()
      pltpu.sync_copy(rows1, acc.at[eidx1.at[1]], add=True)
      pltpu.async_copy(eidx_hbm.at[wid, i + 3], eidx1, semi1)
      pltpu.make_async_copy(eidx_hbm.at[wid, i + 2], eidx0, semi0).wait()
      pltpu.async_copy(h_hbm.at[eidx0.at[0]], rows0, sema)
      pltpu.make_async_copy(eidx_hbm.at[wid, i + 3], eidx1, semi1).wait()
      pltpu.async_copy(h_hbm.at[eidx1.at[0]], rows1, semb)
      return carry

    lax.fori_loop(0, niter // 2 - 1, pair_body, 0)

    pltpu.make_async_copy(h_hbm.at[eidx0.at[0]], rows0, sema).wait()
    pltpu.sync_copy(rows0, acc.at[eidx0.at[1]], add=True)
    pltpu.make_async_copy(h_hbm.at[eidx1.at[0]], rows1, semb).wait()
    pltpu.sync_copy(rows1, acc.at[eidx1.at[1]], add=True)

    plsc.subcore_barrier()
    pltpu.sync_copy(acc.at[pl.ds(row0, rpt)],
                    out_hbm.at[cid, pl.ds(row0, rpt)])

  return segsum


@functools.cache
def _segsum_fn():
  return _make_segsum()


def _segsum(h, eidx):
  return _segsum_fn()(h, eidx)


def kernel(x, edge_index, lin_w, lin_b, conv_w1, conv_w2, bn_gamma, bn_beta):
  pad = _EPAD - E
  src = jnp.concatenate(
      [edge_index[0], jnp.zeros((pad,), jnp.int32)]).reshape(32, _NITER, _CHUNK)
  dst_pad = N + jnp.arange(pad, dtype=jnp.int32) % (_NPAD - N)
  dst = jnp.concatenate([edge_index[1], dst_pad]).reshape(32, _NITER, _CHUNK)
  eidx = jnp.stack([src, dst], axis=2)
  lin_b2 = lin_b.reshape(1, F)
  g2 = bn_gamma.reshape(1, F)
  b2 = bn_beta.reshape(1, F)

  x0 = _compute_x0(x, lin_w, lin_b2)
  h = x0
  for layer, w in enumerate((conv_w1, conv_w2), start=1):
    beta = float(math.log(THETA / layer + 1.0))
    mpart = _segsum(h, eidx)
    h = _dense_layer(mpart, x0, w, g2, b2, beta)
  return h


# speedup vs baseline: 3.0065x; 2.9761x over previous
import functools
import math

import jax
import jax.numpy as jnp
from jax import lax
from jax.experimental import pallas as pl
from jax.experimental.pallas import tpu as pltpu
from jax.experimental.pallas import tpu_sc as plsc

N = 10000
F = 128
E = 320000
_NPAD = 10240
ALPHA = 0.1
THETA = 0.5
EPS = 1e-5

_ROW_BLOCK = 1000
_GRID = N // _ROW_BLOCK


def _x0_body(x_ref, w_ref, b_ref, o_ref):
  acc = lax.dot_general(x_ref[...], w_ref[...], (((1,), (1,)), ((), ())),
                        preferred_element_type=jnp.float32)
  o_ref[...] = jnp.maximum(acc + b_ref[...], 0.0)


def _compute_x0(x, lin_w, lin_b2):
  return pl.pallas_call(
      _x0_body,
      grid=(_GRID,),
      in_specs=[
          pl.BlockSpec((_ROW_BLOCK, F), lambda i: (i, 0)),
          pl.BlockSpec((F, F), lambda i: (0, 0)),
          pl.BlockSpec((1, F), lambda i: (0, 0)),
      ],
      out_specs=pl.BlockSpec((_ROW_BLOCK, F), lambda i: (i, 0)),
      out_shape=jax.ShapeDtypeStruct((N, F), jnp.float32),
  )(x, lin_w, lin_b2)


def _seg_dense_body(m_ref, x0_ref, w_ref, h_ref, s_ref, *, beta):
  m = m_ref[0] + m_ref[1]
  t = (1.0 - ALPHA) * m + ALPHA * x0_ref[...]
  h = (1.0 - beta) * t + beta * jnp.dot(
      t, w_ref[...], preferred_element_type=jnp.float32)
  h_ref[...] = h

  @pl.when(pl.program_id(0) == 0)
  def _():
    s_ref[...] = jnp.zeros_like(s_ref)

  s_ref[0:1, :] = s_ref[0:1, :] + jnp.sum(h, axis=0, keepdims=True)
  s_ref[1:2, :] = s_ref[1:2, :] + jnp.sum(h * h, axis=0, keepdims=True)


def _bn_body(h_ref, s_ref, g_ref, b_ref, o_ref):
  h = h_ref[...]
  mean = s_ref[0:1, :] * (1.0 / N)
  var = s_ref[1:2, :] * (1.0 / N) - mean * mean
  inv = lax.rsqrt(var + EPS)
  o_ref[...] = jnp.maximum((h - mean) * inv * g_ref[...] + b_ref[...], 0.0)


def _dense_layer(mpart, x0, w, g2, b2, beta):
  h, stats = pl.pallas_call(
      functools.partial(_seg_dense_body, beta=beta),
      grid=(_GRID,),
      in_specs=[
          pl.BlockSpec((2, _ROW_BLOCK, F), lambda i: (0, i, 0)),
          pl.BlockSpec((_ROW_BLOCK, F), lambda i: (i, 0)),
          pl.BlockSpec((F, F), lambda i: (0, 0)),
      ],
      out_specs=[
          pl.BlockSpec((_ROW_BLOCK, F), lambda i: (i, 0)),
          pl.BlockSpec((8, F), lambda i: (0, 0)),
      ],
      out_shape=[
          jax.ShapeDtypeStruct((N, F), jnp.float32),
          jax.ShapeDtypeStruct((8, F), jnp.float32),
      ],
  )(mpart, x0, w)
  return pl.pallas_call(
      _bn_body,
      grid=(_GRID,),
      in_specs=[
          pl.BlockSpec((_ROW_BLOCK, F), lambda i: (i, 0)),
          pl.BlockSpec((8, F), lambda i: (0, 0)),
          pl.BlockSpec((1, F), lambda i: (0, 0)),
          pl.BlockSpec((1, F), lambda i: (0, 0)),
      ],
      out_specs=pl.BlockSpec((_ROW_BLOCK, F), lambda i: (i, 0)),
      out_shape=jax.ShapeDtypeStruct((N, F), jnp.float32),
  )(h, stats, g2, b2)


_CHUNK = 128
_NITER = 80
_EPAD = 32 * _NITER * _CHUNK


def _make_segsum():
  info = plsc.get_sparse_core_info()
  nc, ns = info.num_cores, info.num_subcores
  nw = nc * ns
  chunk = _CHUNK
  niter = _NITER
  npad = _NPAD
  rpt = npad // ns

  mesh = plsc.VectorSubcoreMesh(core_axis_name="c", subcore_axis_name="s")

  @functools.partial(
      pl.kernel,
      out_type=jax.ShapeDtypeStruct((nc, npad, F), jnp.float32),
      mesh=mesh,
      scratch_types=[
          pltpu.VMEM((2, chunk), jnp.int32),
          pltpu.VMEM((2, chunk), jnp.int32),
          pltpu.VMEM((chunk, F), jnp.float32),
          pltpu.VMEM((chunk, F), jnp.float32),
          pltpu.VMEM_SHARED((npad, F), jnp.float32),
          pltpu.SemaphoreType.DMA,
          pltpu.SemaphoreType.DMA,
          pltpu.SemaphoreType.DMA,
          pltpu.SemaphoreType.DMA,
      ],
  )
  def segsum(h_hbm, eidx_hbm, out_hbm, eidx0, eidx1, rows0, rows1, acc,
             sema, semb, semi0, semi1):
    cid = lax.axis_index("c")
    sid = lax.axis_index("s")
    wid = sid * nc + cid

    def zero_body(i, carry):
      rows0[i // 8, pl.ds((i % 8) * 16, 16)] = jnp.zeros((16,), jnp.float32)
      return carry

    lax.fori_loop(0, chunk * 8, zero_body, 0)
    row0 = sid * rpt
    for j in range(rpt // chunk):
      pltpu.sync_copy(rows0, acc.at[pl.ds(row0 + j * chunk, chunk)])
    rem = rpt % chunk
    if rem:
      pltpu.sync_copy(rows0.at[pl.ds(0, rem)],
                      acc.at[pl.ds(row0 + rpt - rem, rem)])
    plsc.subcore_barrier()

    pltpu.sync_copy(eidx_hbm.at[wid, 0], eidx0)
    pltpu.async_copy(h_hbm.at[eidx0.at[0]], rows0, sema)
    pltpu.sync_copy(eidx_hbm.at[wid, 1], eidx1)
    pltpu.async_copy(h_hbm.at[eidx1.at[0]], rows1, semb)

    def pair_body(p, carry):
      i = p * 2
      pltpu.make_async_copy(h_hbm.at[eidx0.at[0]], rows0, sema).wait()
      pltpu.sync_copy(rows0, acc.at[eidx0.at[1]], add=True)
      pltpu.async_copy(eidx_hbm.at[wid, i + 2], eidx0, semi0)
      pltpu.make_async_copy(h_hbm.at[eidx1.at[0]], rows1, semb).wait()
      pltpu.sync_copy(rows1, acc.at[eidx1.at[1]], add=True)
      pltpu.async_copy(eidx_hbm.at[wid, i + 3], eidx1, semi1)
      pltpu.make_async_copy(eidx_hbm.at[wid, i + 2], eidx0, semi0).wait()
      pltpu.async_copy(h_hbm.at[eidx0.at[0]], rows0, sema)
      pltpu.make_async_copy(eidx_hbm.at[wid, i + 3], eidx1, semi1).wait()
      pltpu.async_copy(h_hbm.at[eidx1.at[0]], rows1, semb)
      return carry

    lax.fori_loop(0, niter // 2 - 1, pair_body, 0)

    pltpu.make_async_copy(h_hbm.at[eidx0.at[0]], rows0, sema).wait()
    pltpu.sync_copy(rows0, acc.at[eidx0.at[1]], add=True)
    pltpu.make_async_copy(h_hbm.at[eidx1.at[0]], rows1, semb).wait()
    pltpu.sync_copy(rows1, acc.at[eidx1.at[1]], add=True)

    plsc.subcore_barrier()
    pltpu.sync_copy(acc.at[pl.ds(row0, rpt)],
                    out_hbm.at[cid, pl.ds(row0, rpt)])

  return segsum


@functools.cache
def _segsum_fn():
  return _make_segsum()


def _segsum(h, eidx):
  return _segsum_fn()(h, eidx)


def kernel(x, edge_index, lin_w, lin_b, conv_w1, conv_w2, bn_gamma, bn_beta):
  pad = _EPAD - E
  src_pad = jnp.arange(pad, dtype=jnp.int32) % N
  src = jnp.concatenate([edge_index[0], src_pad]).reshape(32, _NITER, _CHUNK)
  dst_pad = N + jnp.arange(pad, dtype=jnp.int32) % (_NPAD - N)
  dst = jnp.concatenate([edge_index[1], dst_pad]).reshape(32, _NITER, _CHUNK)
  eidx = jnp.stack([src, dst], axis=2)
  lin_b2 = lin_b.reshape(1, F)
  g2 = bn_gamma.reshape(1, F)
  b2 = bn_beta.reshape(1, F)

  x0 = _compute_x0(x, lin_w, lin_b2)
  h = x0
  for layer, w in enumerate((conv_w1, conv_w2), start=1):
    beta = float(math.log(THETA / layer + 1.0))
    mpart = _segsum(h, eidx)
    h = _dense_layer(mpart, x0, w, g2, b2, beta)
  return h


# speedup vs baseline: 3.6288x; 1.2070x over previous
import functools
import math

import jax
import jax.numpy as jnp
from jax import lax
from jax.experimental import pallas as pl
from jax.experimental.pallas import tpu as pltpu
from jax.experimental.pallas import tpu_sc as plsc

N = 10000
F = 128
E = 320000
_NPAD = 10240
ALPHA = 0.1
THETA = 0.5
EPS = 1e-5

_ROW_BLOCK = 1000
_GRID = N // _ROW_BLOCK


def _x0_body(x_ref, w_ref, b_ref, o_ref):
  acc = lax.dot_general(x_ref[...], w_ref[...], (((1,), (1,)), ((), ())),
                        preferred_element_type=jnp.float32)
  o_ref[...] = jnp.maximum(acc + b_ref[...], 0.0)


def _compute_x0(x, lin_w, lin_b2):
  return pl.pallas_call(
      _x0_body,
      grid=(_GRID,),
      in_specs=[
          pl.BlockSpec((_ROW_BLOCK, F), lambda i: (i, 0)),
          pl.BlockSpec((F, F), lambda i: (0, 0)),
          pl.BlockSpec((1, F), lambda i: (0, 0)),
      ],
      out_specs=pl.BlockSpec((_ROW_BLOCK, F), lambda i: (i, 0)),
      out_shape=jax.ShapeDtypeStruct((N, F), jnp.float32),
  )(x, lin_w, lin_b2)


def _seg_dense_body(m_ref, x0_ref, w_ref, h_ref, s_ref, *, beta):
  m = m_ref[0] + m_ref[1]
  t = (1.0 - ALPHA) * m + ALPHA * x0_ref[...]
  h = (1.0 - beta) * t + beta * jnp.dot(
      t, w_ref[...], preferred_element_type=jnp.float32)
  h_ref[...] = h

  @pl.when(pl.program_id(0) == 0)
  def _():
    s_ref[...] = jnp.zeros_like(s_ref)

  s_ref[0:1, :] = s_ref[0:1, :] + jnp.sum(h, axis=0, keepdims=True)
  s_ref[1:2, :] = s_ref[1:2, :] + jnp.sum(h * h, axis=0, keepdims=True)


def _bn_body(h_ref, s_ref, g_ref, b_ref, o_ref):
  h = h_ref[...]
  mean = s_ref[0:1, :] * (1.0 / N)
  var = s_ref[1:2, :] * (1.0 / N) - mean * mean
  inv = lax.rsqrt(var + EPS)
  o_ref[...] = jnp.maximum((h - mean) * inv * g_ref[...] + b_ref[...], 0.0)


def _dense_layer(mpart, x0, w, g2, b2, beta):
  h, stats = pl.pallas_call(
      functools.partial(_seg_dense_body, beta=beta),
      grid=(_GRID,),
      in_specs=[
          pl.BlockSpec((2, _ROW_BLOCK, F), lambda i: (0, i, 0)),
          pl.BlockSpec((_ROW_BLOCK, F), lambda i: (i, 0)),
          pl.BlockSpec((F, F), lambda i: (0, 0)),
      ],
      out_specs=[
          pl.BlockSpec((_ROW_BLOCK, F), lambda i: (i, 0)),
          pl.BlockSpec((8, F), lambda i: (0, 0)),
      ],
      out_shape=[
          jax.ShapeDtypeStruct((N, F), jnp.float32),
          jax.ShapeDtypeStruct((8, F), jnp.float32),
      ],
  )(mpart, x0, w)
  return pl.pallas_call(
      _bn_body,
      grid=(_GRID,),
      in_specs=[
          pl.BlockSpec((_ROW_BLOCK, F), lambda i: (i, 0)),
          pl.BlockSpec((8, F), lambda i: (0, 0)),
          pl.BlockSpec((1, F), lambda i: (0, 0)),
          pl.BlockSpec((1, F), lambda i: (0, 0)),
      ],
      out_specs=pl.BlockSpec((_ROW_BLOCK, F), lambda i: (i, 0)),
      out_shape=jax.ShapeDtypeStruct((N, F), jnp.float32),
  )(h, stats, g2, b2)


_CHUNK = 64
_NITER = 160
_EPAD = 32 * _NITER * _CHUNK


def _make_segsum():
  info = plsc.get_sparse_core_info()
  nc, ns = info.num_cores, info.num_subcores
  nw = nc * ns
  chunk = _CHUNK
  niter = _NITER
  npad = _NPAD
  rpt = npad // ns

  mesh = plsc.VectorSubcoreMesh(core_axis_name="c", subcore_axis_name="s")

  @functools.partial(
      pl.kernel,
      out_type=jax.ShapeDtypeStruct((nc, npad, F), jnp.float32),
      mesh=mesh,
      scratch_types=(
          [pltpu.VMEM((2, chunk), jnp.int32)] * 8 +
          [pltpu.VMEM((chunk, F), jnp.float32)] * 4 +
          [pltpu.VMEM_SHARED((npad, F), jnp.float32)] +
          [pltpu.SemaphoreType.DMA] * 14
      ),
  )
  def segsum(h_hbm, eidx_hbm, out_hbm, *sc):
    eidx = list(sc[0:8])
    rows = list(sc[8:12])
    acc = sc[12]
    semg = list(sc[13:17])
    sems = list(sc[17:19])
    semi = list(sc[19:27])

    cid = lax.axis_index("c")
    sid = lax.axis_index("s")
    wid = sid * nc + cid

    def zero_body(i, carry):
      rows[0][i // 8, pl.ds((i % 8) * 16, 16)] = jnp.zeros((16,), jnp.float32)
      return carry

    lax.fori_loop(0, chunk * 8, zero_body, 0)
    row0 = sid * rpt
    for j in range(rpt // chunk):
      pltpu.sync_copy(rows[0], acc.at[pl.ds(row0 + j * chunk, chunk)])
    plsc.subcore_barrier()

    def emit_body(i, ti):
      r, k, s = i % 4, i % 8, i % 2
      if i >= 4:
        pltpu.make_async_copy(
            rows[(i - 4) % 4], acc.at[eidx[(i - 4) % 8].at[1]],
            sems[s]).wait()
      pltpu.make_async_copy(eidx_hbm.at[wid, 0], eidx[k], semi[k]).wait()
      pltpu.async_copy(h_hbm.at[eidx[k].at[0]], rows[r], semg[r])
      if i >= 2:
        r2, k2 = (i - 2) % 4, (i - 2) % 8
        pltpu.make_async_copy(
            h_hbm.at[eidx[k2].at[0]], rows[r2], semg[r2]).wait()
        pltpu.async_copy(rows[r2], acc.at[eidx[k2].at[1]], sems[(i - 2) % 2],
                         add=True)
      if i + 4 <= niter - 1:
        k4 = (i + 4) % 8
        pltpu.async_copy(eidx_hbm.at[wid, ti + 4], eidx[k4], semi[k4])

    for j in range(4):
      pltpu.async_copy(eidx_hbm.at[wid, j], eidx[j], semi[j])
    for i in range(8):
      emit_body(i, i)

    def main_body(q, carry):
      base = 8 + q * 8
      for off in range(8):
        emit_body(8 + off, base + off)
      return carry

    lax.fori_loop(0, (niter - 16) // 8, main_body, 0)

    for i in range(niter - 8, niter):
      emit_body(i, i)

    for i in (niter - 2, niter - 1):
      r2, k2 = i % 4, i % 8
      pltpu.make_async_copy(
          h_hbm.at[eidx[k2].at[0]], rows[r2], semg[r2]).wait()
      pltpu.async_copy(rows[r2], acc.at[eidx[k2].at[1]], sems[i % 2],
                       add=True)
    for i in range(4):
      pltpu.make_async_copy(
          rows[(niter - 4 + i) % 4], acc.at[eidx[(niter - 4 + i) % 8].at[1]],
          sems[i % 2]).wait()

    plsc.subcore_barrier()
    pltpu.sync_copy(acc.at[pl.ds(row0, rpt)],
                    out_hbm.at[cid, pl.ds(row0, rpt)])

  return segsum


@functools.cache
def _segsum_fn():
  return _make_segsum()


def _segsum(h, eidx):
  return _segsum_fn()(h, eidx)


def kernel(x, edge_index, lin_w, lin_b, conv_w1, conv_w2, bn_gamma, bn_beta):
  pad = _EPAD - E
  src_pad = jnp.arange(pad, dtype=jnp.int32) % N
  src = jnp.concatenate([edge_index[0], src_pad]).reshape(32, _NITER, _CHUNK)
  dst_pad = N + jnp.arange(pad, dtype=jnp.int32) % (_NPAD - N)
  dst = jnp.concatenate([edge_index[1], dst_pad]).reshape(32, _NITER, _CHUNK)
  eidx = jnp.stack([src, dst], axis=2)
  lin_b2 = lin_b.reshape(1, F)
  g2 = bn_gamma.reshape(1, F)
  b2 = bn_beta.reshape(1, F)

  x0 = _compute_x0(x, lin_w, lin_b2)
  h = x0
  for layer, w in enumerate((conv_w1, conv_w2), start=1):
    beta = float(math.log(THETA / layer + 1.0))
    mpart = _segsum(h, eidx)
    h = _dense_layer(mpart, x0, w, g2, b2, beta)
  return h


# speedup vs baseline: 3.9220x; 1.0808x over previous
import functools
import math

import jax
import jax.numpy as jnp
from jax import lax
from jax.experimental import pallas as pl
from jax.experimental.pallas import tpu as pltpu
from jax.experimental.pallas import tpu_sc as plsc

N = 10000
F = 128
E = 320000
_NPAD = 10240
ALPHA = 0.1
THETA = 0.5
EPS = 1e-5

_ROW_BLOCK = 2000
_GRID = N // _ROW_BLOCK


def _x0_body(x_ref, w_ref, b_ref, o_ref):
  acc = lax.dot_general(x_ref[...], w_ref[...], (((1,), (1,)), ((), ())),
                        preferred_element_type=jnp.float32)
  o_ref[...] = jnp.maximum(acc + b_ref[...], 0.0)


def _compute_x0(x, lin_w, lin_b2):
  return pl.pallas_call(
      _x0_body,
      grid=(_GRID,),
      in_specs=[
          pl.BlockSpec((_ROW_BLOCK, F), lambda i: (i, 0)),
          pl.BlockSpec((F, F), lambda i: (0, 0)),
          pl.BlockSpec((1, F), lambda i: (0, 0)),
      ],
      out_specs=pl.BlockSpec((_ROW_BLOCK, F), lambda i: (i, 0)),
      out_shape=jax.ShapeDtypeStruct((N, F), jnp.float32),
  )(x, lin_w, lin_b2)


def _seg_dense_body(m_ref, x0_ref, w_ref, h_ref, s_ref, *, beta):
  m = m_ref[0] + m_ref[1]
  t = (1.0 - ALPHA) * m + ALPHA * x0_ref[...]
  h = (1.0 - beta) * t + beta * jnp.dot(
      t, w_ref[...], preferred_element_type=jnp.float32)
  h_ref[...] = h

  @pl.when(pl.program_id(0) == 0)
  def _():
    s_ref[...] = jnp.zeros_like(s_ref)

  s_ref[0:1, :] = s_ref[0:1, :] + jnp.sum(h, axis=0, keepdims=True)
  s_ref[1:2, :] = s_ref[1:2, :] + jnp.sum(h * h, axis=0, keepdims=True)


def _bn_body(h_ref, s_ref, g_ref, b_ref, o_ref):
  h = h_ref[...]
  mean = s_ref[0:1, :] * (1.0 / N)
  var = s_ref[1:2, :] * (1.0 / N) - mean * mean
  inv = lax.rsqrt(var + EPS)
  o_ref[...] = jnp.maximum((h - mean) * inv * g_ref[...] + b_ref[...], 0.0)


def _dense_layer(mpart, x0, w, g2, b2, beta):
  h, stats = pl.pallas_call(
      functools.partial(_seg_dense_body, beta=beta),
      grid=(_GRID,),
      in_specs=[
          pl.BlockSpec((2, _ROW_BLOCK, F), lambda i: (0, i, 0)),
          pl.BlockSpec((_ROW_BLOCK, F), lambda i: (i, 0)),
          pl.BlockSpec((F, F), lambda i: (0, 0)),
      ],
      out_specs=[
          pl.BlockSpec((_ROW_BLOCK, F), lambda i: (i, 0)),
          pl.BlockSpec((8, F), lambda i: (0, 0)),
      ],
      out_shape=[
          jax.ShapeDtypeStruct((N, F), jnp.float32),
          jax.ShapeDtypeStruct((8, F), jnp.float32),
      ],
  )(mpart, x0, w)
  return pl.pallas_call(
      _bn_body,
      grid=(_GRID,),
      in_specs=[
          pl.BlockSpec((_ROW_BLOCK, F), lambda i: (i, 0)),
          pl.BlockSpec((8, F), lambda i: (0, 0)),
          pl.BlockSpec((1, F), lambda i: (0, 0)),
          pl.BlockSpec((1, F), lambda i: (0, 0)),
      ],
      out_specs=pl.BlockSpec((_ROW_BLOCK, F), lambda i: (i, 0)),
      out_shape=jax.ShapeDtypeStruct((N, F), jnp.float32),
  )(h, stats, g2, b2)


_CHUNK = 64
_NITER = 160
_EPAD = 32 * _NITER * _CHUNK


def _make_segsum():
  info = plsc.get_sparse_core_info()
  nc, ns = info.num_cores, info.num_subcores
  nw = nc * ns
  chunk = _CHUNK
  niter = _NITER
  npad = _NPAD
  rpt = npad // ns

  mesh = plsc.VectorSubcoreMesh(core_axis_name="c", subcore_axis_name="s")

  @functools.partial(
      pl.kernel,
      out_type=jax.ShapeDtypeStruct((nc, npad, F), jnp.float32),
      mesh=mesh,
      scratch_types=(
          [pltpu.VMEM((chunk,), jnp.int32)] * 8 +
          [pltpu.VMEM((chunk,), jnp.int32)] * 8 +
          [pltpu.VMEM((chunk, F), jnp.float32)] * 4 +
          [pltpu.VMEM_SHARED((npad, F), jnp.float32)] +
          [pltpu.SemaphoreType.DMA] * 22
      ),
  )
  def segsum(h_hbm, src_hbm, dst_hbm, out_hbm, *sc):
    sidx = list(sc[0:8])
    didx = list(sc[8:16])
    rows = list(sc[16:20])
    acc = sc[20]
    semg = list(sc[21:25])
    sems = list(sc[25:27])
    semi = list(sc[27:35])
    semj = list(sc[35:43])

    cid = lax.axis_index("c")
    sid = lax.axis_index("s")
    wid = sid * nc + cid
    epw = niter * chunk
    ebase = wid * epw

    def zero_body(i, carry):
      rows[0][i // 8, pl.ds((i % 8) * 16, 16)] = jnp.zeros((16,), jnp.float32)
      return carry

    lax.fori_loop(0, chunk * 8, zero_body, 0)
    row0 = sid * rpt
    for j in range(rpt // chunk):
      pltpu.sync_copy(rows[0], acc.at[pl.ds(row0 + j * chunk, chunk)])
    plsc.subcore_barrier()

    def emit_body(i, ti):
      r, k, s = i % 4, i % 8, i % 2
      if i >= 4:
        pltpu.make_async_copy(
            rows[(i - 4) % 4], acc.at[didx[(i - 4) % 8]], sems[s]).wait()
      pltpu.make_async_copy(src_hbm.at[pl.ds(0, chunk)], sidx[k],
                            semi[k]).wait()
      pltpu.make_async_copy(dst_hbm.at[pl.ds(0, chunk)], didx[k],
                            semj[k]).wait()
      pltpu.async_copy(h_hbm.at[sidx[k]], rows[r], semg[r])
      if i >= 2:
        r2, k2 = (i - 2) % 4, (i - 2) % 8
        pltpu.make_async_copy(
            h_hbm.at[sidx[k2]], rows[r2], semg[r2]).wait()
        pltpu.async_copy(rows[r2], acc.at[didx[k2]], sems[(i - 2) % 2],
                         add=True)
      if i + 4 <= niter - 1:
        k4 = (i + 4) % 8
        off4 = ebase + (ti + 4) * chunk
        pltpu.async_copy(src_hbm.at[pl.ds(off4, chunk)], sidx[k4], semi[k4])
        pltpu.async_copy(dst_hbm.at[pl.ds(off4, chunk)], didx[k4], semj[k4])

    for j in range(4):
      pltpu.async_copy(src_hbm.at[pl.ds(ebase + j * chunk, chunk)], sidx[j],
                       semi[j])
      pltpu.async_copy(dst_hbm.at[pl.ds(ebase + j * chunk, chunk)], didx[j],
                       semj[j])
    for i in range(8):
      emit_body(i, i)

    def main_body(q, carry):
      base = 8 + q * 8
      for off in range(8):
        emit_body(8 + off, base + off)
      return carry

    lax.fori_loop(0, (niter - 16) // 8, main_body, 0)

    for i in range(niter - 8, niter):
      emit_body(i, i)

    for i in (niter - 2, niter - 1):
      r2, k2 = i % 4, i % 8
      pltpu.make_async_copy(
          h_hbm.at[sidx[k2]], rows[r2], semg[r2]).wait()
      pltpu.async_copy(rows[r2], acc.at[didx[k2]], sems[i % 2], add=True)
    for i in range(4):
      pltpu.make_async_copy(
          rows[(niter - 4 + i) % 4], acc.at[didx[(niter - 4 + i) % 8]],
          sems[i % 2]).wait()

    plsc.subcore_barrier()
    pltpu.sync_copy(acc.at[pl.ds(row0, rpt)],
                    out_hbm.at[cid, pl.ds(row0, rpt)])

  return segsum


@functools.cache
def _segsum_fn():
  return _make_segsum()


def _segsum(h, src, dst):
  return _segsum_fn()(h, src, dst)


def kernel(x, edge_index, lin_w, lin_b, conv_w1, conv_w2, bn_gamma, bn_beta):
  pad = _EPAD - E
  src = jnp.concatenate(
      [edge_index[0], jnp.arange(pad, dtype=jnp.int32) % N])
  dst = jnp.concatenate(
      [edge_index[1], N + jnp.arange(pad, dtype=jnp.int32) % (_NPAD - N)])
  lin_b2 = lin_b.reshape(1, F)
  g2 = bn_gamma.reshape(1, F)
  b2 = bn_beta.reshape(1, F)

  x0 = _compute_x0(x, lin_w, lin_b2)
  h = x0
  for layer, w in enumerate((conv_w1, conv_w2), start=1):
    beta = float(math.log(THETA / layer + 1.0))
    mpart = _segsum(h, src, dst)
    h = _dense_layer(mpart, x0, w, g2, b2, beta)
  return h


# speedup vs baseline: 4.0760x; 1.0393x over previous
import functools
import math

import jax
import jax.numpy as jnp
from jax import lax
from jax.experimental import pallas as pl
from jax.experimental.pallas import tpu as pltpu
from jax.experimental.pallas import tpu_sc as plsc

N = 10000
F = 128
E = 320000
_NPAD = 10240
ALPHA = 0.1
THETA = 0.5
EPS = 1e-5

_ROW_BLOCK = 2000
_GRID = N // _ROW_BLOCK


def _x0_body(x_ref, w_ref, b_ref, o_ref):
  acc = lax.dot_general(x_ref[...], w_ref[...], (((1,), (1,)), ((), ())),
                        preferred_element_type=jnp.float32)
  o_ref[...] = jnp.maximum(acc + b_ref[...], 0.0)


def _compute_x0(x, lin_w, lin_b2):
  return pl.pallas_call(
      _x0_body,
      grid=(_GRID,),
      in_specs=[
          pl.BlockSpec((_ROW_BLOCK, F), lambda i: (i, 0)),
          pl.BlockSpec((F, F), lambda i: (0, 0)),
          pl.BlockSpec((1, F), lambda i: (0, 0)),
      ],
      out_specs=pl.BlockSpec((_ROW_BLOCK, F), lambda i: (i, 0)),
      out_shape=jax.ShapeDtypeStruct((N, F), jnp.float32),
  )(x, lin_w, lin_b2)


def _layer_body(m_ref, x0_ref, w_ref, g_ref, b_ref, o_ref, h_scr, s_scr, *,
                beta):
  p = pl.program_id(0)
  i = pl.program_id(1)

  @pl.when(p == 0)
  def _():
    m = m_ref[0] + m_ref[1]
    t = (1.0 - ALPHA) * m + ALPHA * x0_ref[...]
    h = (1.0 - beta) * t + beta * jnp.dot(
        t, w_ref[...], preferred_element_type=jnp.float32)
    h_scr[pl.ds(i * _ROW_BLOCK, _ROW_BLOCK), :] = h

    @pl.when(i == 0)
    def _():
      s_scr[...] = jnp.zeros_like(s_scr)

    s_scr[0:1, :] = s_scr[0:1, :] + jnp.sum(h, axis=0, keepdims=True)
    s_scr[1:2, :] = s_scr[1:2, :] + jnp.sum(h * h, axis=0, keepdims=True)

  @pl.when(p == 1)
  def _():
    h = h_scr[pl.ds(i * _ROW_BLOCK, _ROW_BLOCK), :]
    mean = s_scr[0:1, :] * (1.0 / N)
    var = s_scr[1:2, :] * (1.0 / N) - mean * mean
    inv = lax.rsqrt(var + EPS)
    o_ref[...] = jnp.maximum((h - mean) * inv * g_ref[...] + b_ref[...], 0.0)


def _dense_layer(mpart, x0, w, g2, b2, beta):
  return pl.pallas_call(
      functools.partial(_layer_body, beta=beta),
      grid=(2, _GRID),
      in_specs=[
          pl.BlockSpec((2, _ROW_BLOCK, F),
                       lambda p, i: (0, jnp.where(p == 0, i, _GRID - 1), 0)),
          pl.BlockSpec((_ROW_BLOCK, F),
                       lambda p, i: (jnp.where(p == 0, i, _GRID - 1), 0)),
          pl.BlockSpec((F, F), lambda p, i: (0, 0)),
          pl.BlockSpec((1, F), lambda p, i: (0, 0)),
          pl.BlockSpec((1, F), lambda p, i: (0, 0)),
      ],
      out_specs=pl.BlockSpec((_ROW_BLOCK, F),
                             lambda p, i: (jnp.where(p == 0, 0, i), 0)),
      out_shape=jax.ShapeDtypeStruct((N, F), jnp.float32),
      scratch_shapes=[
          pltpu.VMEM((N, F), jnp.float32),
          pltpu.VMEM((8, F), jnp.float32),
      ],
  )(mpart, x0, w, g2, b2)


_CHUNK = 64
_NITER = 160
_EPAD = 32 * _NITER * _CHUNK


def _make_segsum():
  info = plsc.get_sparse_core_info()
  nc, ns = info.num_cores, info.num_subcores
  nw = nc * ns
  chunk = _CHUNK
  niter = _NITER
  npad = _NPAD
  rpt = npad // ns

  mesh = plsc.VectorSubcoreMesh(core_axis_name="c", subcore_axis_name="s")

  @functools.partial(
      pl.kernel,
      out_type=jax.ShapeDtypeStruct((nc, npad, F), jnp.float32),
      mesh=mesh,
      scratch_types=(
          [pltpu.VMEM((chunk,), jnp.int32)] * 8 +
          [pltpu.VMEM((chunk,), jnp.int32)] * 8 +
          [pltpu.VMEM((chunk, F), jnp.float32)] * 4 +
          [pltpu.VMEM_SHARED((npad, F), jnp.float32)] +
          [pltpu.SemaphoreType.DMA] * 22
      ),
  )
  def segsum(h_hbm, src_hbm, dst_hbm, out_hbm, *sc):
    sidx = list(sc[0:8])
    didx = list(sc[8:16])
    rows = list(sc[16:20])
    acc = sc[20]
    semg = list(sc[21:25])
    sems = list(sc[25:27])
    semi = list(sc[27:35])
    semj = list(sc[35:43])

    cid = lax.axis_index("c")
    sid = lax.axis_index("s")
    wid = sid * nc + cid
    epw = niter * chunk
    ebase = wid * epw

    def zero_body(i, carry):
      rows[0][i // 8, pl.ds((i % 8) * 16, 16)] = jnp.zeros((16,), jnp.float32)
      return carry

    lax.fori_loop(0, chunk * 8, zero_body, 0)
    row0 = sid * rpt
    for j in range(rpt // chunk):
      pltpu.sync_copy(rows[0], acc.at[pl.ds(row0 + j * chunk, chunk)])
    plsc.subcore_barrier()

    def emit_body(i, ti):
      r, k, s = i % 4, i % 8, i % 2
      if i >= 4:
        pltpu.make_async_copy(
            rows[(i - 4) % 4], acc.at[didx[(i - 4) % 8]], sems[s]).wait()
      pltpu.make_async_copy(src_hbm.at[pl.ds(0, chunk)], sidx[k],
                            semi[k]).wait()
      pltpu.make_async_copy(dst_hbm.at[pl.ds(0, chunk)], didx[k],
                            semj[k]).wait()
      pltpu.async_copy(h_hbm.at[sidx[k]], rows[r], semg[r])
      if i >= 2:
        r2, k2 = (i - 2) % 4, (i - 2) % 8
        pltpu.make_async_copy(
            h_hbm.at[sidx[k2]], rows[r2], semg[r2]).wait()
        pltpu.async_copy(rows[r2], acc.at[didx[k2]], sems[(i - 2) % 2],
                         add=True)
      if i + 4 <= niter - 1:
        k4 = (i + 4) % 8
        off4 = ebase + (ti + 4) * chunk
        pltpu.async_copy(src_hbm.at[pl.ds(off4, chunk)], sidx[k4], semi[k4])
        pltpu.async_copy(dst_hbm.at[pl.ds(off4, chunk)], didx[k4], semj[k4])

    for j in range(4):
      pltpu.async_copy(src_hbm.at[pl.ds(ebase + j * chunk, chunk)], sidx[j],
                       semi[j])
      pltpu.async_copy(dst_hbm.at[pl.ds(ebase + j * chunk, chunk)], didx[j],
                       semj[j])
    for i in range(8):
      emit_body(i, i)

    def main_body(q, carry):
      base = 8 + q * 8
      for off in range(8):
        emit_body(8 + off, base + off)
      return carry

    lax.fori_loop(0, (niter - 16) // 8, main_body, 0)

    for i in range(niter - 8, niter):
      emit_body(i, i)

    for i in (niter - 2, niter - 1):
      r2, k2 = i % 4, i % 8
      pltpu.make_async_copy(
          h_hbm.at[sidx[k2]], rows[r2], semg[r2]).wait()
      pltpu.async_copy(rows[r2], acc.at[didx[k2]], sems[i % 2], add=True)
    for i in range(4):
      pltpu.make_async_copy(
          rows[(niter - 4 + i) % 4], acc.at[didx[(niter - 4 + i) % 8]],
          sems[i % 2]).wait()

    plsc.subcore_barrier()
    pltpu.sync_copy(acc.at[pl.ds(row0, rpt)],
                    out_hbm.at[cid, pl.ds(row0, rpt)])

  return segsum


@functools.cache
def _segsum_fn():
  return _make_segsum()


def _segsum(h, src, dst):
  return _segsum_fn()(h, src, dst)


def kernel(x, edge_index, lin_w, lin_b, conv_w1, conv_w2, bn_gamma, bn_beta):
  pad = _EPAD - E
  src = jnp.concatenate(
      [edge_index[0], jnp.arange(pad, dtype=jnp.int32) % N])
  dst = jnp.concatenate(
      [edge_index[1], N + jnp.arange(pad, dtype=jnp.int32) % (_NPAD - N)])
  lin_b2 = lin_b.reshape(1, F)
  g2 = bn_gamma.reshape(1, F)
  b2 = bn_beta.reshape(1, F)

  x0 = _compute_x0(x, lin_w, lin_b2)
  h = x0
  for layer, w in enumerate((conv_w1, conv_w2), start=1):
    beta = float(math.log(THETA / layer + 1.0))
    mpart = _segsum(h, src, dst)
    h = _dense_layer(mpart, x0, w, g2, b2, beta)
  return h


# speedup vs baseline: 4.1513x; 1.0185x over previous
import functools
import math

import jax
import jax.numpy as jnp
from jax import lax
from jax.experimental import pallas as pl
from jax.experimental.pallas import tpu as pltpu
from jax.experimental.pallas import tpu_sc as plsc

N = 10000
F = 128
E = 320000
_NPAD = 10240
ALPHA = 0.1
THETA = 0.5
EPS = 1e-5

_ROW_BLOCK = 2000
_GRID = N // _ROW_BLOCK


def _x0_body(x_ref, w_ref, b_ref, o_ref):
  acc = lax.dot_general(x_ref[...], w_ref[...], (((1,), (1,)), ((), ())),
                        preferred_element_type=jnp.float32)
  o_ref[...] = jnp.maximum(acc + b_ref[...], 0.0)


def _compute_x0(x, lin_w, lin_b2):
  return pl.pallas_call(
      _x0_body,
      grid=(_GRID,),
      in_specs=[
          pl.BlockSpec((_ROW_BLOCK, F), lambda i: (i, 0)),
          pl.BlockSpec((F, F), lambda i: (0, 0)),
          pl.BlockSpec((1, F), lambda i: (0, 0)),
      ],
      out_specs=pl.BlockSpec((_ROW_BLOCK, F), lambda i: (i, 0)),
      out_shape=jax.ShapeDtypeStruct((N, F), jnp.float32),
  )(x, lin_w, lin_b2)


def _layer_body(m_ref, x0_ref, w_ref, g_ref, b_ref, o_ref, h_scr, s_scr, *,
                beta):
  p = pl.program_id(0)
  i = pl.program_id(1)

  @pl.when(p == 0)
  def _():
    m = m_ref[0] + m_ref[1]
    t = (1.0 - ALPHA) * m + ALPHA * x0_ref[...]
    h = (1.0 - beta) * t + beta * jnp.dot(
        t, w_ref[...], preferred_element_type=jnp.float32)
    h_scr[pl.ds(i * _ROW_BLOCK, _ROW_BLOCK), :] = h

    @pl.when(i == 0)
    def _():
      s_scr[...] = jnp.zeros_like(s_scr)

    s_scr[0:1, :] = s_scr[0:1, :] + jnp.sum(h, axis=0, keepdims=True)
    s_scr[1:2, :] = s_scr[1:2, :] + jnp.sum(h * h, axis=0, keepdims=True)

  @pl.when(p == 1)
  def _():
    h = h_scr[pl.ds(i * _ROW_BLOCK, _ROW_BLOCK), :]
    mean = s_scr[0:1, :] * (1.0 / N)
    var = s_scr[1:2, :] * (1.0 / N) - mean * mean
    inv = lax.rsqrt(var + EPS)
    o_ref[...] = jnp.maximum((h - mean) * inv * g_ref[...] + b_ref[...], 0.0)


def _dense_layer(mpart, x0, w, g2, b2, beta):
  return pl.pallas_call(
      functools.partial(_layer_body, beta=beta),
      grid=(2, _GRID),
      in_specs=[
          pl.BlockSpec((2, _ROW_BLOCK, F),
                       lambda p, i: (0, jnp.where(p == 0, i, _GRID - 1), 0)),
          pl.BlockSpec((_ROW_BLOCK, F),
                       lambda p, i: (jnp.where(p == 0, i, _GRID - 1), 0)),
          pl.BlockSpec((F, F), lambda p, i: (0, 0)),
          pl.BlockSpec((1, F), lambda p, i: (0, 0)),
          pl.BlockSpec((1, F), lambda p, i: (0, 0)),
      ],
      out_specs=pl.BlockSpec((_ROW_BLOCK, F),
                             lambda p, i: (jnp.where(p == 0, 0, i), 0)),
      out_shape=jax.ShapeDtypeStruct((N, F), jnp.float32),
      scratch_shapes=[
          pltpu.VMEM((N, F), jnp.float32),
          pltpu.VMEM((8, F), jnp.float32),
      ],
  )(mpart, x0, w, g2, b2)


_CHUNK = 64
_NITER = 160
_EPAD = 32 * _NITER * _CHUNK


def _make_segsum():
  info = plsc.get_sparse_core_info()
  nc, ns = info.num_cores, info.num_subcores
  nw = nc * ns
  chunk = _CHUNK
  niter = _NITER
  npad = _NPAD
  rpt = npad // ns

  mesh = plsc.VectorSubcoreMesh(core_axis_name="c", subcore_axis_name="s")

  @functools.partial(
      pl.kernel,
      out_type=jax.ShapeDtypeStruct((nc, npad, F), jnp.float32),
      mesh=mesh,
      scratch_types=(
          [pltpu.VMEM((chunk,), jnp.int32)] * 8 +
          [pltpu.VMEM((chunk,), jnp.int32)] * 8 +
          [pltpu.VMEM((chunk, F), jnp.float32)] * 4 +
          [pltpu.VMEM_SHARED((npad, F), jnp.float32)] +
          [pltpu.SemaphoreType.DMA] * 22
      ),
  )
  def segsum(h_hbm, src_hbm, dst_hbm, out_hbm, *sc):
    sidx = list(sc[0:8])
    didx = list(sc[8:16])
    rows = list(sc[16:20])
    acc = sc[20]
    semg = list(sc[21:25])
    sems = list(sc[25:27])
    semi = list(sc[27:35])
    semj = list(sc[35:43])

    cid = lax.axis_index("c")
    sid = lax.axis_index("s")
    wid = sid * nc + cid
    epw = niter * chunk
    ebase = wid * epw

    for j in range(4):
      pltpu.async_copy(src_hbm.at[pl.ds(ebase + j * chunk, chunk)], sidx[j],
                       semi[j])
      pltpu.async_copy(dst_hbm.at[pl.ds(ebase + j * chunk, chunk)], didx[j],
                       semj[j])

    def zero_body(i, carry):
      rows[0][i // 8, pl.ds((i % 8) * 16, 16)] = jnp.zeros((16,), jnp.float32)
      return carry

    lax.fori_loop(0, chunk * 8, zero_body, 0)
    row0 = sid * rpt
    for j in range(rpt // chunk):
      pltpu.sync_copy(rows[0], acc.at[pl.ds(row0 + j * chunk, chunk)])
    plsc.subcore_barrier()

    def emit_body(i, ti):
      r, k, s = i % 4, i % 8, i % 2
      if i >= 4:
        pltpu.make_async_copy(
            rows[(i - 4) % 4], acc.at[didx[(i - 4) % 8]], sems[s]).wait()
      pltpu.make_async_copy(src_hbm.at[pl.ds(0, chunk)], sidx[k],
                            semi[k]).wait()
      pltpu.make_async_copy(dst_hbm.at[pl.ds(0, chunk)], didx[k],
                            semj[k]).wait()
      pltpu.async_copy(h_hbm.at[sidx[k]], rows[r], semg[r])
      if i >= 2:
        r2, k2 = (i - 2) % 4, (i - 2) % 8
        pltpu.make_async_copy(
            h_hbm.at[sidx[k2]], rows[r2], semg[r2]).wait()
        pltpu.async_copy(rows[r2], acc.at[didx[k2]], sems[(i - 2) % 2],
                         add=True)
      if i + 4 <= niter - 1:
        k4 = (i + 4) % 8
        off4 = ebase + (ti + 4) * chunk
        pltpu.async_copy(src_hbm.at[pl.ds(off4, chunk)], sidx[k4], semi[k4])
        pltpu.async_copy(dst_hbm.at[pl.ds(off4, chunk)], didx[k4], semj[k4])

    for i in range(8):
      emit_body(i, i)

    def main_body(q, carry):
      base = 8 + q * 8
      for off in range(8):
        emit_body(8 + off, base + off)
      return carry

    lax.fori_loop(0, (niter - 16) // 8, main_body, 0)

    for i in range(niter - 8, niter):
      emit_body(i, i)

    for i in (niter - 2, niter - 1):
      r2, k2 = i % 4, i % 8
      pltpu.make_async_copy(
          h_hbm.at[sidx[k2]], rows[r2], semg[r2]).wait()
      pltpu.async_copy(rows[r2], acc.at[didx[k2]], sems[i % 2], add=True)
    for i in range(4):
      pltpu.make_async_copy(
          rows[(niter - 4 + i) % 4], acc.at[didx[(niter - 4 + i) % 8]],
          sems[i % 2]).wait()

    plsc.subcore_barrier()
    pltpu.sync_copy(acc.at[pl.ds(row0, rpt)],
                    out_hbm.at[cid, pl.ds(row0, rpt)])

  return segsum


@functools.cache
def _segsum_fn():
  return _make_segsum()


def _segsum(h, src, dst):
  return _segsum_fn()(h, src, dst)


def kernel(x, edge_index, lin_w, lin_b, conv_w1, conv_w2, bn_gamma, bn_beta):
  pad = _EPAD - E
  tail = jnp.stack([jnp.arange(pad, dtype=jnp.int32) % N,
                    N + jnp.arange(pad, dtype=jnp.int32) % (_NPAD - N)])
  edges = jnp.concatenate([edge_index, tail], axis=1)
  src = edges[0]
  dst = edges[1]
  lin_b2 = lin_b.reshape(1, F)
  g2 = bn_gamma.reshape(1, F)
  b2 = bn_beta.reshape(1, F)

  x0 = _compute_x0(x, lin_w, lin_b2)
  h = x0
  for layer, w in enumerate((conv_w1, conv_w2), start=1):
    beta = float(math.log(THETA / layer + 1.0))
    mpart = _segsum(h, src, dst)
    h = _dense_layer(mpart, x0, w, g2, b2, beta)
  return h
